# baseline (speedup 1.0000x reference)
import functools

import jax
import jax.numpy as jnp
from jax import lax
from jax.experimental import pallas as pl
from jax.experimental.pallas import tpu as pltpu

MARGIN_ = 0.2
BLK = 2048
NCLS = 128


def _triplet_kernel(e_ref, labr_ref, labc_ref, out_ref, normed_ref, tab_ref,
                    g_ref, acc_ref):
    i = pl.program_id(0)
    nblk = pl.num_programs(0)
    n = e_ref.shape[0]
    inf = jnp.float32(jnp.inf)
    ninf = jnp.float32(-jnp.inf)

    @pl.when(i == 0)
    def _init():
        e = e_ref[:, :]
        norm = jnp.sqrt(jnp.sum(e * e, axis=1, keepdims=True))
        normed = e / jnp.maximum(norm, 1e-12)
        normed_ref[:, :] = normed
        lab_col0 = labc_ref[:, :]
        cls_row0 = lax.broadcasted_iota(jnp.int32, (1, NCLS), 1)
        match = lab_col0 == cls_row0
        j_col0 = lax.broadcasted_iota(jnp.int32, (n, NCLS), 0)
        first = jnp.min(jnp.where(match, j_col0, n), axis=0, keepdims=True)
        second = jnp.min(jnp.where(match & (j_col0 != first), j_col0, n),
                         axis=0, keepdims=True)
        sub = lax.broadcasted_iota(jnp.int32, (8, NCLS), 0)
        tab_ref[:, :] = (jnp.where(sub == 0, first.astype(jnp.float32), 0.0)
                         + jnp.where(sub == 1, second.astype(jnp.float32), 0.0))
        cls_colw = lax.broadcasted_iota(jnp.int32, (NCLS, n), 0)
        j_roww = lax.broadcasted_iota(jnp.int32, (NCLS, n), 1)
        matchw = cls_colw == lab_col0.reshape(1, n)
        first_c = jnp.min(jnp.where(matchw, j_roww, n), axis=1, keepdims=True)
        second_c = jnp.min(jnp.where(matchw & (j_roww != first_c), j_roww, n),
                           axis=1, keepdims=True)
        fs_col = jnp.concatenate([first_c, second_c], axis=0)
        j_row2 = lax.broadcasted_iota(jnp.int32, (2 * NCLS, n), 1)
        onehot_fs = (j_row2 == fs_col).astype(jnp.bfloat16)
        g_ref[:, :] = lax.dot_general(
            onehot_fs, normed.astype(jnp.bfloat16),
            (((1,), (0,)), ((), ())),
            preferred_element_type=jnp.float32,
            precision=lax.Precision.DEFAULT,
        )
        acc_ref[0] = 0.0
        acc_ref[1] = 0.0

    e_all = normed_ref[:, :]
    a = normed_ref[pl.ds(i * BLK, BLK), :]
    sim = lax.dot_general(
        e_all, a, (((1,), (1,)), ((), ())),
        preferred_element_type=jnp.float32,
        precision=lax.Precision.DEFAULT,
    )

    lab_col = labc_ref[:, :]
    lab_a = labr_ref[0, pl.ds(i * BLK, BLK)].reshape(1, BLK)

    cls_col = lax.broadcasted_iota(jnp.int32, (NCLS, 1), 0)
    onehot = (cls_col == lab_a).astype(jnp.float32)
    lk = lax.dot_general(
        tab_ref[:, :], onehot, (((1,), (0,)), ((), ())),
        preferred_element_type=jnp.float32,
        precision=lax.Precision.HIGHEST,
    )
    first_a = lk[0:1, :].astype(jnp.int32)
    second_a = lk[1:2, :].astype(jnp.int32)
    row_i = lax.broadcasted_iota(jnp.int32, (1, BLK), 1) + i * BLK
    is_first = first_a == row_i
    has_pos = (~is_first) | (second_a < n)

    sp = lax.dot_general(
        g_ref[:, :], a, (((1,), (1,)), ((), ())),
        preferred_element_type=jnp.float32,
        precision=lax.Precision.DEFAULT,
    )
    poscls = jnp.where(is_first, lab_a + NCLS, lab_a)
    sub2 = lax.broadcasted_iota(jnp.int32, (2 * NCLS, BLK), 0)
    s_ap = jnp.max(jnp.where(sub2 == poscls, sp, ninf), axis=0, keepdims=True)
    d_ap = jnp.clip(1.0 - s_ap, 0.0, 2.0)

    binf = jnp.bfloat16(jnp.inf)
    ms = jnp.where(lab_col.astype(jnp.bfloat16) == lab_a.astype(jnp.bfloat16),
                   -binf, sim.astype(jnp.bfloat16))
    s_hi = 1.0 - d_ap
    s_lo = s_hi - MARGIN_
    semi = (ms < s_hi.astype(jnp.bfloat16)) & (ms > s_lo.astype(jnp.bfloat16))
    mw = jnp.where(semi, ms, -binf)
    fold = 8
    seg = n // fold
    mh_h = ms[0:seg, :]
    mw_h = mw[0:seg, :]
    for k in range(1, fold):
        mh_h = jnp.maximum(mh_h, ms[k * seg:(k + 1) * seg, :])
        mw_h = jnp.maximum(mw_h, mw[k * seg:(k + 1) * seg, :])
    s_hard = jnp.max(mh_h, axis=0, keepdims=True).astype(jnp.float32)
    s_semi = jnp.max(mw_h, axis=0, keepdims=True).astype(jnp.float32)
    s0 = sim[0:1, :].astype(jnp.float32)
    d_an = jnp.clip(1.0 - jnp.where(s_semi > ninf, s_semi,
                                    jnp.where(s_hard > ninf, s_hard, s0)),
                    0.0, 2.0)

    valid = has_pos.astype(jnp.float32)
    loss = jnp.maximum(d_ap - d_an + MARGIN_, 0.0) * valid

    acc_ref[0] += jnp.sum(loss)
    acc_ref[1] += jnp.sum(valid)

    @pl.when(i == nblk - 1)
    def _finish():
        cnt = acc_ref[1]
        mean = acc_ref[0] / jnp.maximum(cnt, 1.0)
        out_ref[0, 0] = jnp.where(cnt > 0.0, mean, 0.0)


def _build_call(n, d, interpret=False):
    return pl.pallas_call(
        _triplet_kernel,
        grid=(n // BLK,),
        in_specs=[
            pl.BlockSpec((n, d), lambda i: (0, 0)),
            pl.BlockSpec((1, n), lambda i: (0, 0)),
            pl.BlockSpec((n, 1), lambda i: (0, 0)),
        ],
        out_specs=pl.BlockSpec((1, 1), lambda i: (0, 0), memory_space=pltpu.SMEM),
        out_shape=jax.ShapeDtypeStruct((1, 1), jnp.float32),
        scratch_shapes=[
            pltpu.VMEM((n, d), jnp.float32),
            pltpu.VMEM((8, NCLS), jnp.float32),
            pltpu.VMEM((2 * NCLS, d), jnp.float32),
            pltpu.SMEM((2,), jnp.float32),
        ],
        interpret=interpret,
    )


def kernel(embeddings, labels):
    n, d = embeddings.shape
    lab = labels.astype(jnp.int32)
    out = _build_call(n, d)(embeddings, lab.reshape(1, n), lab.reshape(n, 1))
    return out.reshape(())


# speedup vs baseline: 1.0880x; 1.0880x over previous
import functools

import jax
import jax.numpy as jnp
from jax import lax
from jax.experimental import pallas as pl
from jax.experimental.pallas import tpu as pltpu

MARGIN_ = 0.2
BLK = 2048
NCLS = 128


def _triplet_kernel(e_ref, labr_ref, labc_ref, out_ref, normed_ref, tab_ref,
                    g_ref, acc_ref):
    i = pl.program_id(0)
    nblk = pl.num_programs(0)
    n = e_ref.shape[0]
    inf = jnp.float32(jnp.inf)
    ninf = jnp.float32(-jnp.inf)

    @pl.when(i == 0)
    def _init():
        e = e_ref[:, :]
        norm = jnp.sqrt(jnp.sum(e * e, axis=1, keepdims=True))
        normed = e / jnp.maximum(norm, 1e-12)
        normed_ref[:, :] = normed
        lab_col0 = labc_ref[:, :]
        cls_row0 = lax.broadcasted_iota(jnp.int32, (1, NCLS), 1)
        match = lab_col0 == cls_row0
        j_col0 = lax.broadcasted_iota(jnp.int32, (n, NCLS), 0)
        first = jnp.min(jnp.where(match, j_col0, n), axis=0, keepdims=True)
        second = jnp.min(jnp.where(match & (j_col0 != first), j_col0, n),
                         axis=0, keepdims=True)
        sub = lax.broadcasted_iota(jnp.int32, (8, NCLS), 0)
        tab_ref[:, :] = (jnp.where(sub == 0, first.astype(jnp.float32), 0.0)
                         + jnp.where(sub == 1, second.astype(jnp.float32), 0.0))
        cls_colw = lax.broadcasted_iota(jnp.int32, (NCLS, n), 0)
        j_roww = lax.broadcasted_iota(jnp.int32, (NCLS, n), 1)
        matchw = cls_colw == lab_col0.reshape(1, n)
        first_c = jnp.min(jnp.where(matchw, j_roww, n), axis=1, keepdims=True)
        second_c = jnp.min(jnp.where(matchw & (j_roww != first_c), j_roww, n),
                           axis=1, keepdims=True)
        fs_col = jnp.concatenate([first_c, second_c], axis=0)
        j_row2 = lax.broadcasted_iota(jnp.int32, (2 * NCLS, n), 1)
        onehot_fs = (j_row2 == fs_col).astype(jnp.bfloat16)
        g_ref[:, :] = lax.dot_general(
            onehot_fs, normed.astype(jnp.bfloat16),
            (((1,), (0,)), ((), ())),
            preferred_element_type=jnp.float32,
            precision=lax.Precision.DEFAULT,
        )
        acc_ref[0] = 0.0
        acc_ref[1] = 0.0

    e_all = normed_ref[:, :]
    a = normed_ref[pl.ds(i * BLK, BLK), :]
    sim = lax.dot_general(
        e_all, a, (((1,), (1,)), ((), ())),
        preferred_element_type=jnp.float32,
        precision=lax.Precision.DEFAULT,
    )

    lab_col = labc_ref[:, :]
    lab_a = labr_ref[0, pl.ds(i * BLK, BLK)].reshape(1, BLK)

    cls_col = lax.broadcasted_iota(jnp.int32, (NCLS, 1), 0)
    onehot = (cls_col == lab_a).astype(jnp.float32)
    lk = lax.dot_general(
        tab_ref[:, :], onehot, (((1,), (0,)), ((), ())),
        preferred_element_type=jnp.float32,
        precision=lax.Precision.HIGHEST,
    )
    first_a = lk[0:1, :].astype(jnp.int32)
    second_a = lk[1:2, :].astype(jnp.int32)
    row_i = lax.broadcasted_iota(jnp.int32, (1, BLK), 1) + i * BLK
    is_first = first_a == row_i
    has_pos = (~is_first) | (second_a < n)

    sp = lax.dot_general(
        g_ref[:, :], a, (((1,), (1,)), ((), ())),
        preferred_element_type=jnp.float32,
        precision=lax.Precision.DEFAULT,
    )
    poscls = jnp.where(is_first, lab_a + NCLS, lab_a)
    sub2 = lax.broadcasted_iota(jnp.int32, (2 * NCLS, BLK), 0)
    s_ap = jnp.max(jnp.where(sub2 == poscls, sp, ninf), axis=0, keepdims=True)
    d_ap = jnp.clip(1.0 - s_ap, 0.0, 2.0)

    binf = jnp.bfloat16(jnp.inf)
    ms = jnp.where(lab_col.astype(jnp.bfloat16) == lab_a.astype(jnp.bfloat16),
                   -binf, sim.astype(jnp.bfloat16))
    s_hi = 1.0 - d_ap
    s_lo = s_hi - MARGIN_
    semi = (ms < s_hi.astype(jnp.bfloat16)) & (ms > s_lo.astype(jnp.bfloat16))
    s_hard = jnp.max(ms, axis=0, keepdims=True).astype(jnp.float32)
    s_semi = jnp.max(jnp.where(semi, ms, -binf), axis=0,
                     keepdims=True).astype(jnp.float32)
    s0 = sim[0:1, :].astype(jnp.float32)
    d_an = jnp.clip(1.0 - jnp.where(s_semi > ninf, s_semi,
                                    jnp.where(s_hard > ninf, s_hard, s0)),
                    0.0, 2.0)

    valid = has_pos.astype(jnp.float32)
    loss = jnp.maximum(d_ap - d_an + MARGIN_, 0.0) * valid

    acc_ref[0] += jnp.sum(loss)
    acc_ref[1] += jnp.sum(valid)

    @pl.when(i == nblk - 1)
    def _finish():
        cnt = acc_ref[1]
        mean = acc_ref[0] / jnp.maximum(cnt, 1.0)
        out_ref[0, 0] = jnp.where(cnt > 0.0, mean, 0.0)


def _build_call(n, d, interpret=False):
    return pl.pallas_call(
        _triplet_kernel,
        grid=(n // BLK,),
        in_specs=[
            pl.BlockSpec((n, d), lambda i: (0, 0)),
            pl.BlockSpec((1, n), lambda i: (0, 0)),
            pl.BlockSpec((n, 1), lambda i: (0, 0)),
        ],
        out_specs=pl.BlockSpec((1, 1), lambda i: (0, 0), memory_space=pltpu.SMEM),
        out_shape=jax.ShapeDtypeStruct((1, 1), jnp.float32),
        scratch_shapes=[
            pltpu.VMEM((n, d), jnp.float32),
            pltpu.VMEM((8, NCLS), jnp.float32),
            pltpu.VMEM((2 * NCLS, d), jnp.float32),
            pltpu.SMEM((2,), jnp.float32),
        ],
        interpret=interpret,
    )


def kernel(embeddings, labels):
    n, d = embeddings.shape
    lab = labels.astype(jnp.int32)
    out = _build_call(n, d)(embeddings, lab.reshape(1, n), lab.reshape(n, 1))
    return out.reshape(())


# speedup vs baseline: 1.0904x; 1.0022x over previous
import jax
import jax.numpy as jnp
from jax import lax
from jax.experimental import pallas as pl
from jax.experimental.pallas import tpu as pltpu

MARGIN_ = 0.2
BLK = 2048
NCLS = 128


def _triplet_kernel(e_ref, labr_ref, labc_ref, out_ref, normed_ref, tab_ref,
                    g_ref, acc_ref):
    i = pl.program_id(0)
    nblk = pl.num_programs(0)
    n = e_ref.shape[0]
    ninf = jnp.float32(-jnp.inf)

    @pl.when(i == 0)
    def _init():
        e = e_ref[:, :]
        norm = jnp.sqrt(jnp.sum(e * e, axis=1, keepdims=True))
        normed = e / jnp.maximum(norm, 1e-12)
        normed_ref[:, :] = normed
        lab_col0 = labc_ref[:, :]
        cls_row0 = lax.broadcasted_iota(jnp.int32, (1, NCLS), 1)
        match = lab_col0 == cls_row0
        j_col0 = lax.broadcasted_iota(jnp.int32, (n, NCLS), 0)
        first = jnp.min(jnp.where(match, j_col0, n), axis=0, keepdims=True)
        second = jnp.min(jnp.where(match & (j_col0 != first), j_col0, n),
                         axis=0, keepdims=True)
        sub = lax.broadcasted_iota(jnp.int32, (8, NCLS), 0)
        tab_ref[:, :] = (jnp.where(sub == 0, first.astype(jnp.float32), 0.0)
                         + jnp.where(sub == 1, second.astype(jnp.float32), 0.0))
        cls_colw = lax.broadcasted_iota(jnp.int32, (NCLS, n), 0)
        j_roww = lax.broadcasted_iota(jnp.int32, (NCLS, n), 1)
        matchw = cls_colw == lab_col0.reshape(1, n)
        first_c = jnp.min(jnp.where(matchw, j_roww, n), axis=1, keepdims=True)
        second_c = jnp.min(jnp.where(matchw & (j_roww != first_c), j_roww, n),
                           axis=1, keepdims=True)
        fs_col = jnp.concatenate([first_c, second_c], axis=0)
        j_row2 = lax.broadcasted_iota(jnp.int32, (2 * NCLS, n), 1)
        onehot_fs = (j_row2 == fs_col).astype(jnp.bfloat16)
        g_ref[:, :] = lax.dot_general(
            onehot_fs, normed.astype(jnp.bfloat16),
            (((1,), (0,)), ((), ())),
            preferred_element_type=jnp.float32,
            precision=lax.Precision.DEFAULT,
        )
        acc_ref[0] = 0.0
        acc_ref[1] = 0.0

    e_all = normed_ref[:, :]
    a = normed_ref[pl.ds(i * BLK, BLK), :]
    sim = lax.dot_general(
        e_all, a, (((1,), (1,)), ((), ())),
        preferred_element_type=jnp.float32,
        precision=lax.Precision.DEFAULT,
    )

    lab_col = labc_ref[:, :]
    lab_a = labr_ref[0, pl.ds(i * BLK, BLK)].reshape(1, BLK)

    cls_col = lax.broadcasted_iota(jnp.int32, (NCLS, 1), 0)
    onehot = (cls_col == lab_a).astype(jnp.float32)
    lk = lax.dot_general(
        tab_ref[:, :], onehot, (((1,), (0,)), ((), ())),
        preferred_element_type=jnp.float32,
        precision=lax.Precision.HIGHEST,
    )
    first_a = lk[0:1, :].astype(jnp.int32)
    second_a = lk[1:2, :].astype(jnp.int32)
    row_i = lax.broadcasted_iota(jnp.int32, (1, BLK), 1) + i * BLK
    is_first = first_a == row_i
    has_pos = (~is_first) | (second_a < n)

    sp = lax.dot_general(
        g_ref[:, :], a, (((1,), (1,)), ((), ())),
        preferred_element_type=jnp.float32,
        precision=lax.Precision.DEFAULT,
    )
    poscls = jnp.where(is_first, lab_a + NCLS, lab_a)
    sub2 = lax.broadcasted_iota(jnp.int32, (2 * NCLS, BLK), 0)
    s_ap = jnp.max(jnp.where(sub2 == poscls, sp, ninf), axis=0, keepdims=True)
    d_ap = jnp.clip(1.0 - s_ap, 0.0, 2.0)

    binf = jnp.bfloat16(jnp.inf)
    ms = jnp.where(lab_col.astype(jnp.bfloat16) == lab_a.astype(jnp.bfloat16),
                   -binf, sim.astype(jnp.bfloat16))
    s_hi = 1.0 - d_ap
    s_lo = s_hi - MARGIN_
    semi = (ms < s_hi.astype(jnp.bfloat16)) & (ms > s_lo.astype(jnp.bfloat16))
    s_hard = jnp.max(ms, axis=0, keepdims=True).astype(jnp.float32)
    s_semi = jnp.max(jnp.where(semi, ms, -binf), axis=0,
                     keepdims=True).astype(jnp.float32)
    s0 = sim[0:1, :].astype(jnp.float32)
    d_an = jnp.clip(1.0 - jnp.where(s_semi > ninf, s_semi,
                                    jnp.where(s_hard > ninf, s_hard, s0)),
                    0.0, 2.0)

    valid = has_pos.astype(jnp.float32)
    loss = jnp.maximum(d_ap - d_an + MARGIN_, 0.0) * valid

    acc_ref[0] += jnp.sum(loss)
    acc_ref[1] += jnp.sum(valid)

    @pl.when(i == nblk - 1)
    def _finish():
        cnt = acc_ref[1]
        mean = acc_ref[0] / jnp.maximum(cnt, 1.0)
        out_ref[0, 0] = jnp.where(cnt > 0.0, mean, 0.0)


def _build_call(n, d, interpret=False):
    return pl.pallas_call(
        _triplet_kernel,
        grid=(n // BLK,),
        in_specs=[
            pl.BlockSpec((n, d), lambda i: (0, 0)),
            pl.BlockSpec((1, n), lambda i: (0, 0)),
            pl.BlockSpec((n, 1), lambda i: (0, 0)),
        ],
        out_specs=pl.BlockSpec((1, 1), lambda i: (0, 0), memory_space=pltpu.SMEM),
        out_shape=jax.ShapeDtypeStruct((1, 1), jnp.float32),
        scratch_shapes=[
            pltpu.VMEM((n, d), jnp.float32),
            pltpu.VMEM((8, NCLS), jnp.float32),
            pltpu.VMEM((2 * NCLS, d), jnp.float32),
            pltpu.SMEM((2,), jnp.float32),
        ],
        interpret=interpret,
    )


def kernel(embeddings, labels):
    n, d = embeddings.shape
    lab = labels.astype(jnp.int32)
    out = _build_call(n, d)(embeddings, lab.reshape(1, n), lab.reshape(n, 1))
    return out.reshape(())
